# baseline (speedup 1.0000x reference)
import jax
import jax.numpy as jnp
from jax.experimental import pallas as pl
from jax.experimental.pallas import tpu as pltpu

_H = 64
_W = 64
_PIX = _H * _W
_PAD = 128


def _fused_block(x_ref, m_ref, w1_ref, w2_ref, w3_ref, o_ref, yp_ref):
    xg = x_ref[0]
    m = m_ref[0]
    w1 = w1_ref[0]
    w3 = w3_ref[0]

    xm = xg * m
    y1 = jnp.maximum(jnp.dot(w1, xm, preferred_element_type=jnp.float32), 0.0)
    y1b = y1.astype(jnp.bfloat16)

    yp_ref[:, :_PAD] = jnp.zeros((64, _PAD), jnp.bfloat16)
    yp_ref[:, _PAD + _PIX:] = jnp.zeros((64, _PAD), jnp.bfloat16)
    yp_ref[:, _PAD:_PAD + _PIX] = y1b

    col = jax.lax.broadcasted_iota(jnp.int32, (1, _PIX), 1)
    wpos = jnp.bitwise_and(col, _W - 1)
    left_ok = (wpos > 0).astype(jnp.bfloat16)
    right_ok = (wpos < _W - 1).astype(jnp.bfloat16)

    acc = jnp.zeros((64, _PIX), jnp.float32)
    for kh in range(3):
        for kw in range(3):
            s = (kh - 1) * _W + (kw - 1)
            z = yp_ref[:, _PAD + s:_PAD + s + _PIX]
            if kw == 0:
                z = z * left_ok
            elif kw == 2:
                z = z * right_ok
            acc = acc + jnp.dot(w2_ref[0, kh * 3 + kw], z,
                                preferred_element_type=jnp.float32)
    y2 = jnp.maximum(acc, 0.0).astype(jnp.bfloat16)
    y3 = jnp.dot(w3, y2, preferred_element_type=jnp.float32)
    res = jnp.maximum(xg.astype(jnp.float32) + y3 * m.astype(jnp.float32), 0.0)
    o_ref[0] = res.astype(jnp.bfloat16)


def kernel(x, mask, w1, w2, w3):
    b, c, h, w = x.shape
    g = mask.shape[1]
    cg = c // g
    og = w3.shape[0] // g
    mid = w1.shape[0] // g

    xbf = x.astype(jnp.bfloat16).reshape(g, cg, _PIX)
    mh = mask.shape[2]
    mpix = jnp.repeat(jnp.repeat(mask[0], h // mh, axis=1),
                      w // mask.shape[3], axis=2)
    mpix = mpix.reshape(g, 1, _PIX).astype(jnp.bfloat16)
    w1r = w1.reshape(g, mid, cg).astype(jnp.bfloat16)
    w2r = jnp.transpose(w2.reshape(g, mid, mid, 9),
                        (0, 3, 1, 2)).astype(jnp.bfloat16)
    w3r = w3.reshape(g, og, mid).astype(jnp.bfloat16)

    out = pl.pallas_call(
        _fused_block,
        grid=(g,),
        in_specs=[
            pl.BlockSpec((1, cg, _PIX), lambda i: (i, 0, 0)),
            pl.BlockSpec((1, 1, _PIX), lambda i: (i, 0, 0)),
            pl.BlockSpec((1, mid, cg), lambda i: (i, 0, 0)),
            pl.BlockSpec((1, 9, mid, mid), lambda i: (i, 0, 0, 0)),
            pl.BlockSpec((1, og, mid), lambda i: (i, 0, 0)),
        ],
        out_specs=pl.BlockSpec((1, og, _PIX), lambda i: (i, 0, 0)),
        out_shape=jax.ShapeDtypeStruct((g, og, _PIX), jnp.bfloat16),
        scratch_shapes=[pltpu.VMEM((mid, _PIX + 2 * _PAD), jnp.bfloat16)],
    )(xbf, mpix, w1r, w2r, w3r)
    return out.astype(jnp.float32).reshape(b, c, h, w)


# speedup vs baseline: 1.1740x; 1.1740x over previous
import jax
import jax.numpy as jnp
from jax.experimental import pallas as pl
from jax.experimental.pallas import tpu as pltpu

_H = 64
_W = 64
_PIX = _H * _W
_PAD = 128
_CHK = 128


def _fused_block(x_ref, m_ref, w1_ref, w2_ref, w3_ref, o_ref,
                 xflat_ref, y1_ref, yp_ref, y2_ref):
    t = pl.program_id(1)
    m = m_ref[0]
    w1 = w1_ref[0]
    w3 = w3_ref[0]

    @pl.when(t == 0)
    def _():
        xc = x_ref[0].reshape(_CHK, _PIX)
        xflat_ref[0:_CHK, :] = xc
        y1_ref[...] = jnp.dot(w1[:, 0:_CHK], xc * m,
                              preferred_element_type=jnp.float32)

    @pl.when(t == 1)
    def _():
        xc = x_ref[0].reshape(_CHK, _PIX)
        xflat_ref[_CHK:2 * _CHK, :] = xc
        y1_ref[...] += jnp.dot(w1[:, _CHK:2 * _CHK], xc * m,
                               preferred_element_type=jnp.float32)

    @pl.when(t == 2)
    def _():
        y1 = jnp.maximum(y1_ref[...], 0.0).astype(jnp.bfloat16)
        yp_ref[:, :_PAD] = jnp.zeros((64, _PAD), jnp.bfloat16)
        yp_ref[:, _PAD + _PIX:] = jnp.zeros((64, _PAD), jnp.bfloat16)
        yp_ref[:, _PAD:_PAD + _PIX] = y1

        col = jax.lax.broadcasted_iota(jnp.int32, (1, _PIX), 1)
        wpos = jnp.bitwise_and(col, _W - 1)
        left_ok = (wpos > 0).astype(jnp.bfloat16)
        right_ok = (wpos < _W - 1).astype(jnp.bfloat16)

        acc = jnp.zeros((64, _PIX), jnp.float32)
        for kh in range(3):
            for kw in range(3):
                s = (kh - 1) * _W + (kw - 1)
                z = yp_ref[:, _PAD + s:_PAD + s + _PIX]
                if kw == 0:
                    z = z * left_ok
                elif kw == 2:
                    z = z * right_ok
                acc = acc + jnp.dot(w2_ref[0, kh * 3 + kw], z,
                                    preferred_element_type=jnp.float32)
        y2 = jnp.maximum(acc, 0.0).astype(jnp.bfloat16)
        y2_ref[...] = y2
        y3 = jnp.dot(w3[0:_CHK], y2, preferred_element_type=jnp.float32)
        res = jnp.maximum(
            xflat_ref[0:_CHK, :].astype(jnp.float32)
            + y3 * m.astype(jnp.float32), 0.0)
        o_ref[0] = res.astype(jnp.bfloat16).reshape(_CHK, _H, _W)

    @pl.when(t == 3)
    def _():
        y3 = jnp.dot(w3[_CHK:2 * _CHK], y2_ref[...],
                     preferred_element_type=jnp.float32)
        res = jnp.maximum(
            xflat_ref[_CHK:2 * _CHK, :].astype(jnp.float32)
            + y3 * m.astype(jnp.float32), 0.0)
        o_ref[0] = res.astype(jnp.bfloat16).reshape(_CHK, _H, _W)


def kernel(x, mask, w1, w2, w3):
    b, c, h, w = x.shape
    g = mask.shape[1]
    cg = c // g
    og = w3.shape[0] // g
    mid = w1.shape[0] // g

    xbf = x.astype(jnp.bfloat16)
    mh = mask.shape[2]
    mpix = jnp.repeat(jnp.repeat(mask[0], h // mh, axis=1),
                      w // mask.shape[3], axis=2)
    mpix = mpix.reshape(g, 1, _PIX).astype(jnp.bfloat16)
    w1r = w1.reshape(g, mid, cg).astype(jnp.bfloat16)
    w2r = jnp.transpose(w2.reshape(g, mid, mid, 9),
                        (0, 3, 1, 2)).astype(jnp.bfloat16)
    w3r = w3.reshape(g, og, mid).astype(jnp.bfloat16)

    out = pl.pallas_call(
        _fused_block,
        grid=(g, 4),
        in_specs=[
            pl.BlockSpec((1, _CHK, h, w),
                         lambda i, t: (0, 2 * i + jnp.minimum(t, 1), 0, 0)),
            pl.BlockSpec((1, 1, _PIX), lambda i, t: (i, 0, 0)),
            pl.BlockSpec((1, mid, cg), lambda i, t: (i, 0, 0)),
            pl.BlockSpec((1, 9, mid, mid), lambda i, t: (i, 0, 0, 0)),
            pl.BlockSpec((1, og, mid), lambda i, t: (i, 0, 0)),
        ],
        out_specs=pl.BlockSpec(
            (1, _CHK, h, w),
            lambda i, t: (0, 2 * i + jnp.maximum(t - 2, 0), 0, 0)),
        out_shape=jax.ShapeDtypeStruct((b, c, h, w), jnp.bfloat16),
        scratch_shapes=[
            pltpu.VMEM((cg, _PIX), jnp.bfloat16),
            pltpu.VMEM((mid, _PIX), jnp.float32),
            pltpu.VMEM((mid, _PIX + 2 * _PAD), jnp.bfloat16),
            pltpu.VMEM((mid, _PIX), jnp.bfloat16),
        ],
    )(xbf, mpix, w1r, w2r, w3r)
    return out.astype(jnp.float32)


# speedup vs baseline: 1.1766x; 1.0022x over previous
import jax
import jax.numpy as jnp
from jax.experimental import pallas as pl
from jax.experimental.pallas import tpu as pltpu

_H = 64
_W = 64
_PIX = _H * _W
_PAD = 128
_CHK = 128


def _fused_block(x_ref, m_ref, w1_ref, w2_ref, w3_ref, o_ref,
                 xflat_ref, y1_ref, yp_ref, y2_ref):
    t = pl.program_id(1)
    m = m_ref[0]
    w1 = w1_ref[0].astype(jnp.bfloat16)
    w3 = w3_ref[0].astype(jnp.bfloat16)

    @pl.when(t == 0)
    def _():
        xc = x_ref[0].reshape(_CHK, _PIX)
        xflat_ref[0:_CHK, :] = xc
        y1_ref[...] = jnp.dot(w1[:, 0:_CHK], xc * m,
                              preferred_element_type=jnp.float32)

    @pl.when(t == 1)
    def _():
        xc = x_ref[0].reshape(_CHK, _PIX)
        xflat_ref[_CHK:2 * _CHK, :] = xc
        y1_ref[...] += jnp.dot(w1[:, _CHK:2 * _CHK], xc * m,
                               preferred_element_type=jnp.float32)

    @pl.when(t == 2)
    def _():
        y1 = jnp.maximum(y1_ref[...], 0.0).astype(jnp.bfloat16)
        yp_ref[:, :_PAD] = jnp.zeros((64, _PAD), jnp.bfloat16)
        yp_ref[:, _PAD + _PIX:] = jnp.zeros((64, _PAD), jnp.bfloat16)
        yp_ref[:, _PAD:_PAD + _PIX] = y1

        col = jax.lax.broadcasted_iota(jnp.int32, (1, _PIX), 1)
        wpos = jnp.bitwise_and(col, _W - 1)
        left_ok = (wpos > 0).astype(jnp.bfloat16)
        right_ok = (wpos < _W - 1).astype(jnp.bfloat16)

        acc = jnp.zeros((64, _PIX), jnp.float32)
        for kh in range(3):
            for kw in range(3):
                s = (kh - 1) * _W + (kw - 1)
                z = yp_ref[:, _PAD + s:_PAD + s + _PIX]
                if kw == 0:
                    z = z * left_ok
                elif kw == 2:
                    z = z * right_ok
                acc = acc + jnp.dot(w2_ref[0, kh * 3 + kw], z,
                                    preferred_element_type=jnp.float32)
        y2 = jnp.maximum(acc, 0.0).astype(jnp.bfloat16)
        y2_ref[...] = y2
        y3 = jnp.dot(w3[0:_CHK], y2, preferred_element_type=jnp.float32)
        res = jnp.maximum(
            xflat_ref[0:_CHK, :].astype(jnp.float32)
            + y3 * m.astype(jnp.float32), 0.0)
        o_ref[0] = res.astype(jnp.bfloat16).reshape(_CHK, _H, _W)

    @pl.when(t == 3)
    def _():
        y3 = jnp.dot(w3[_CHK:2 * _CHK], y2_ref[...],
                     preferred_element_type=jnp.float32)
        res = jnp.maximum(
            xflat_ref[_CHK:2 * _CHK, :].astype(jnp.float32)
            + y3 * m.astype(jnp.float32), 0.0)
        o_ref[0] = res.astype(jnp.bfloat16).reshape(_CHK, _H, _W)


def kernel(x, mask, w1, w2, w3):
    b, c, h, w = x.shape
    g = mask.shape[1]
    cg = c // g
    og = w3.shape[0] // g
    mid = w1.shape[0] // g

    xbf = x.astype(jnp.bfloat16)
    mh = mask.shape[2]
    mpix = jnp.repeat(jnp.repeat(mask[0], h // mh, axis=1),
                      w // mask.shape[3], axis=2)
    mpix = mpix.reshape(g, 1, _PIX).astype(jnp.bfloat16)
    w1r = w1.reshape(g, mid, cg)
    w2r = jnp.transpose(w2.reshape(g, mid, mid, 9),
                        (0, 3, 1, 2)).astype(jnp.bfloat16)
    w3r = w3.reshape(g, og, mid)

    out = pl.pallas_call(
        _fused_block,
        grid=(g, 4),
        in_specs=[
            pl.BlockSpec((1, _CHK, h, w),
                         lambda i, t: (0, 2 * i + jnp.minimum(t, 1), 0, 0)),
            pl.BlockSpec((1, 1, _PIX), lambda i, t: (i, 0, 0)),
            pl.BlockSpec((1, mid, cg), lambda i, t: (i, 0, 0)),
            pl.BlockSpec((1, 9, mid, mid), lambda i, t: (i, 0, 0, 0)),
            pl.BlockSpec((1, og, mid), lambda i, t: (i, 0, 0)),
        ],
        out_specs=pl.BlockSpec(
            (1, _CHK, h, w),
            lambda i, t: (0, 2 * i + jnp.maximum(t - 2, 0), 0, 0)),
        out_shape=jax.ShapeDtypeStruct((b, c, h, w), jnp.bfloat16),
        scratch_shapes=[
            pltpu.VMEM((cg, _PIX), jnp.bfloat16),
            pltpu.VMEM((mid, _PIX), jnp.float32),
            pltpu.VMEM((mid, _PIX + 2 * _PAD), jnp.bfloat16),
            pltpu.VMEM((mid, _PIX), jnp.bfloat16),
        ],
    )(xbf, mpix, w1r, w2r, w3r)
    return out.astype(jnp.float32)


# speedup vs baseline: 1.2173x; 1.0345x over previous
import jax
import jax.numpy as jnp
from jax.experimental import pallas as pl
from jax.experimental.pallas import tpu as pltpu

_H = 64
_W = 64
_PIX = _H * _W
_PAD = 128


def _fused_block(x_ref, m_ref, w1_ref, w2_ref, w3_ref, o_ref, yp_ref):
    xg = x_ref[0].reshape(x_ref.shape[1], _PIX)
    m = m_ref[0]
    w1 = w1_ref[0].astype(jnp.bfloat16)
    w3 = w3_ref[0].astype(jnp.bfloat16)

    xm = xg * m
    y1 = jnp.maximum(jnp.dot(w1, xm, preferred_element_type=jnp.float32), 0.0)
    y1b = y1.astype(jnp.bfloat16)

    yp_ref[:, :_PAD] = jnp.zeros((64, _PAD), jnp.bfloat16)
    yp_ref[:, _PAD + _PIX:] = jnp.zeros((64, _PAD), jnp.bfloat16)
    yp_ref[:, _PAD:_PAD + _PIX] = y1b

    col = jax.lax.broadcasted_iota(jnp.int32, (1, _PIX), 1)
    wpos = jnp.bitwise_and(col, _W - 1)
    left_ok = (wpos > 0).astype(jnp.bfloat16)
    right_ok = (wpos < _W - 1).astype(jnp.bfloat16)

    acc = jnp.zeros((64, _PIX), jnp.float32)
    for kh in range(3):
        for kw in range(3):
            s = (kh - 1) * _W + (kw - 1)
            z = yp_ref[:, _PAD + s:_PAD + s + _PIX]
            if kw == 0:
                z = z * left_ok
            elif kw == 2:
                z = z * right_ok
            acc = acc + jnp.dot(w2_ref[0, kh * 3 + kw], z,
                                preferred_element_type=jnp.float32)
    y2 = jnp.maximum(acc, 0.0).astype(jnp.bfloat16)
    y3 = jnp.dot(w3, y2, preferred_element_type=jnp.float32)
    res = jnp.maximum(xg.astype(jnp.float32) + y3 * m.astype(jnp.float32), 0.0)
    o_ref[0] = res.astype(jnp.bfloat16).reshape(o_ref.shape[1], _H, _W)


def kernel(x, mask, w1, w2, w3):
    b, c, h, w = x.shape
    g = mask.shape[1]
    cg = c // g
    og = w3.shape[0] // g
    mid = w1.shape[0] // g

    xbf = x.astype(jnp.bfloat16)
    mh = mask.shape[2]
    mpix = jnp.repeat(jnp.repeat(mask[0], h // mh, axis=1),
                      w // mask.shape[3], axis=2)
    mpix = mpix.reshape(g, 1, _PIX).astype(jnp.bfloat16)
    w1r = w1.reshape(g, mid, cg)
    w2r = jnp.transpose(w2.reshape(g, mid, mid, 9),
                        (0, 3, 1, 2)).astype(jnp.bfloat16)
    w3r = w3.reshape(g, og, mid)

    out = pl.pallas_call(
        _fused_block,
        grid=(g,),
        in_specs=[
            pl.BlockSpec((1, cg, h, w), lambda i: (0, i, 0, 0)),
            pl.BlockSpec((1, 1, _PIX), lambda i: (i, 0, 0)),
            pl.BlockSpec((1, mid, cg), lambda i: (i, 0, 0)),
            pl.BlockSpec((1, 9, mid, mid), lambda i: (i, 0, 0, 0)),
            pl.BlockSpec((1, og, mid), lambda i: (i, 0, 0)),
        ],
        out_specs=pl.BlockSpec((1, og, h, w), lambda i: (0, i, 0, 0)),
        out_shape=jax.ShapeDtypeStruct((b, c, h, w), jnp.bfloat16),
        scratch_shapes=[pltpu.VMEM((mid, _PIX + 2 * _PAD), jnp.bfloat16)],
    )(xbf, mpix, w1r, w2r, w3r)
    return out.astype(jnp.float32)
